# baseline (speedup 1.0000x reference)
import functools

import jax
import jax.numpy as jnp
from jax import lax
from jax.experimental import pallas as pl
from jax.experimental.pallas import tpu as pltpu
from jax.experimental.pallas import tpu_sc as plsc

_B = 16384
_K = 16
_INFO = plsc.get_sparse_core_info()
_NC = _INFO.num_cores
_NS = _INFO.num_subcores
_NW = _NC * _NS
_BPW = _B // _NW
_CHUNK = 128
_NCHUNK = _BPW // _CHUNK

_mesh = plsc.VectorSubcoreMesh(core_axis_name="c", subcore_axis_name="s")


@functools.partial(
    pl.kernel,
    mesh=_mesh,
    compiler_params=pltpu.CompilerParams(
        needs_layout_passes=False, use_tc_tiling_on_sc=False),
    out_type=jax.ShapeDtypeStruct((_B,), jnp.float32),
    scratch_types=[
        pltpu.VMEM((_NCHUNK, _CHUNK), jnp.int32),
        pltpu.VMEM((_NCHUNK, _CHUNK), jnp.int32),
        pltpu.VMEM((_BPW, _K), jnp.float32),
        pltpu.VMEM((_BPW, _K), jnp.float32),
        pltpu.VMEM((_BPW,), jnp.float32),
        pltpu.SemaphoreType.DMA,
    ],
)
def _mf_dot(uidx_hbm, iidx_hbm, w_hbm, h_hbm, out_hbm,
            uidx_v, iidx_v, u_v, v_v, o_v, sem):
    wid = lax.axis_index("s") * _NC + lax.axis_index("c")
    base = wid * _BPW

    pltpu.sync_copy(uidx_hbm.at[pl.ds(wid * _NCHUNK, _NCHUNK)], uidx_v)
    pltpu.sync_copy(iidx_hbm.at[pl.ds(wid * _NCHUNK, _NCHUNK)], iidx_v)

    copies = []
    for c in range(_NCHUNK):
        copies.append(pltpu.async_copy(
            w_hbm.at[uidx_v.at[c]], u_v.at[pl.ds(c * _CHUNK, _CHUNK)], sem))
        copies.append(pltpu.async_copy(
            h_hbm.at[iidx_v.at[c]], v_v.at[pl.ds(c * _CHUNK, _CHUNK)], sem))
    for cp in copies:
        cp.wait()

    def body(g, carry):
        lane = lax.iota(jnp.int32, 16)
        acc = jnp.zeros((16,), jnp.float32)
        for j in range(16):
            row = g * 16 + j
            prod = u_v[row] * v_v[row]
            acc = jnp.where(lane == j, jnp.sum(prod), acc)
        o_v[pl.ds(g * 16, 16)] = acc
        return carry

    lax.fori_loop(0, _BPW // 16, body, 0)

    pltpu.sync_copy(o_v, out_hbm.at[pl.ds(base, _BPW)])


def kernel(x, W, H):
    x = x.astype(jnp.int32)
    uidx = x[:, 0].reshape(_B // _CHUNK, _CHUNK)
    iidx = x[:, 1].reshape(_B // _CHUNK, _CHUNK)
    return _mf_dot(uidx, iidx, W, H)
